# baseline (speedup 1.0000x reference)
import jax
import jax.numpy as jnp
from jax.experimental import pallas as pl
from jax.experimental.pallas import tpu as pltpu

B = 16
T = 4
P = 1025
H = 1280
R = 9
PB = 205


def _body(ids_ref, gate_ref, h_ref, emb_ref, o_ref):
    g = jnp.tanh(gate_ref[0])
    row = ids_ref[pl.program_id(0)]
    for t in range(T):
        e = emb_ref[row, pl.ds(t * H, H)]
        o_ref[0, :, t, :] = h_ref[0, :, t, :] + e * g


def kernel(hidden_state, aspect_ratio_ids, embedding_weight, gate):
    ids = aspect_ratio_ids.astype(jnp.int32)
    hp = jnp.transpose(hidden_state, (0, 2, 1, 3))

    grid_spec = pltpu.PrefetchScalarGridSpec(
        num_scalar_prefetch=2,
        grid=(B, P // PB),
        in_specs=[
            pl.BlockSpec((1, PB, T, H), lambda b, p, ids, gate: (b, p, 0, 0)),
            pl.BlockSpec((R, T * H), lambda b, p, ids, gate: (0, 0)),
        ],
        out_specs=pl.BlockSpec((1, PB, T, H), lambda b, p, ids, gate: (b, p, 0, 0)),
    )

    out = pl.pallas_call(
        _body,
        grid_spec=grid_spec,
        out_shape=jax.ShapeDtypeStruct((B, P, T, H), jnp.float32),
    )(ids, gate, hp, embedding_weight)
    return jnp.transpose(out, (0, 2, 1, 3))


# speedup vs baseline: 1.0306x; 1.0306x over previous
import jax
import jax.numpy as jnp
from jax.experimental import pallas as pl
from jax.experimental.pallas import tpu as pltpu

B = 16
T = 4
P = 1025
H = 1280
R = 9
PB = 205
BB = 2


def _body(ids_ref, gate_ref, h_ref, emb_ref, o_ref, tab):
    bi = pl.program_id(0)

    @pl.when(jnp.logical_and(bi == 0, pl.program_id(1) == 0))
    def _():
        for t in range(T):
            tab[:, t, :] = emb_ref[:, pl.ds(t * H, H)]

    g = jnp.tanh(gate_ref[0])
    for db in range(BB):
        row = ids_ref[BB * bi + db]
        o_ref[db] = h_ref[db] + tab[row] * g


def kernel(hidden_state, aspect_ratio_ids, embedding_weight, gate):
    ids = aspect_ratio_ids.astype(jnp.int32)
    hp = jnp.transpose(hidden_state, (0, 2, 1, 3))

    grid_spec = pltpu.PrefetchScalarGridSpec(
        num_scalar_prefetch=2,
        grid=(B // BB, P // PB),
        in_specs=[
            pl.BlockSpec((BB, PB, T, H), lambda b, p, ids, gate: (b, p, 0, 0)),
            pl.BlockSpec((R, T * H), lambda b, p, ids, gate: (0, 0)),
        ],
        out_specs=pl.BlockSpec((BB, PB, T, H), lambda b, p, ids, gate: (b, p, 0, 0)),
        scratch_shapes=[pltpu.VMEM((R, T, H), jnp.float32)],
    )

    out = pl.pallas_call(
        _body,
        grid_spec=grid_spec,
        out_shape=jax.ShapeDtypeStruct((B, P, T, H), jnp.float32),
    )(ids, gate, hp, embedding_weight)
    return jnp.transpose(out, (0, 2, 1, 3))
